# Initial kernel scaffold; baseline (speedup 1.0000x reference)
import jax
import jax.numpy as jnp
from jax.experimental import pallas as pl


def kernel(time_start, initial_state_params):
    raise NotImplementedError("write your pallas kernel here")


# speedup vs baseline: 419.5391x; 419.5391x over previous
import functools

import jax
import jax.numpy as jnp
from jax import lax
from jax.experimental import pallas as pl
from jax.experimental.pallas import tpu as pltpu
from jax.experimental.pallas import tpu_sc as plsc

_PERIOD = 168
_BATCH = 65536
_NC = 2
_NS = 16
_NW = _NC * _NS
_DPAD = 176
_ROWS_PER_W = _BATCH // _NW
_CHUNK = 128
_NCHUNK = _ROWS_PER_W // _CHUNK
_TAB_ROWS_PER_W = 6


def _mesh():
    return plsc.VectorSubcoreMesh(core_axis_name="c", subcore_axis_name="s")


def _build_table(params):
    @functools.partial(
        pl.kernel,
        out_type=jax.ShapeDtypeStruct((_PERIOD, _DPAD), jnp.float32),
        mesh=_mesh(),
        compiler_params=pltpu.CompilerParams(
            needs_layout_passes=False, use_tc_tiling_on_sc=False
        ),
        scratch_types=[
            pltpu.VMEM((_DPAD,), jnp.float32),
            pltpu.VMEM((_DPAD,), jnp.float32),
        ],
    )
    def tab_kernel(params_hbm, table_hbm, params_v, row_v):
        wid = lax.axis_index("s") * _NC + lax.axis_index("c")
        zeros = jnp.zeros((16,), jnp.float32)
        for c in range(_DPAD // 16):
            params_v[pl.ds(c * 16, 16)] = zeros
        pltpu.sync_copy(params_hbm, params_v.at[pl.ds(0, _PERIOD - 1)])
        acc = zeros
        for c in range(_DPAD // 16):
            acc = acc + params_v[pl.ds(c * 16, 16)]
        lanes = lax.iota(jnp.int32, 16)
        for st in (8, 4, 2, 1):
            row_v[pl.ds(0, 16)] = acc
            acc = acc + plsc.load_gather(row_v, [jnp.bitwise_xor(lanes, st)])
        first = -acc
        for r in range(_TAB_ROWS_PER_W):
            m = wid * _TAB_ROWS_PER_W + r

            @pl.when(m < _PERIOD)
            def _():
                for c in range(_DPAD // 16):
                    j = lax.rem(m + c * 16 + lanes, _PERIOD)
                    pidx = jnp.maximum(j - 1, 0)
                    vals = plsc.load_gather(params_v, [pidx])
                    row_v[pl.ds(c * 16, 16)] = jnp.where(j == 0, first, vals)
                pltpu.sync_copy(row_v, table_hbm.at[m])

    return tab_kernel(params)


def _gather_rows(table, ts2):
    @functools.partial(
        pl.kernel,
        out_type=jax.ShapeDtypeStruct((_BATCH, _PERIOD), jnp.float32),
        mesh=_mesh(),
        compiler_params=pltpu.CompilerParams(
            needs_layout_passes=False, use_tc_tiling_on_sc=False
        ),
        scratch_types=[
            pltpu.VMEM((_ROWS_PER_W,), jnp.int32),
            pltpu.VMEM((_CHUNK, _DPAD), jnp.float32),
            pltpu.VMEM((_CHUNK, _DPAD), jnp.float32),
            pltpu.SemaphoreType.DMA,
            pltpu.SemaphoreType.DMA,
        ],
    )
    def gather_kernel(table_hbm, ts_hbm, out_hbm, idx_v, buf0, buf1, sem0, sem1):
        wid = lax.axis_index("s") * _NC + lax.axis_index("c")
        row0 = wid * _ROWS_PER_W
        pltpu.sync_copy(ts_hbm.at[pl.ds(row0, _ROWS_PER_W)], idx_v)

        def mod_body(i, carry):
            v = idx_v[pl.ds(i * 16, 16)]
            idx_v[pl.ds(i * 16, 16)] = lax.rem(v, _PERIOD)
            return carry

        lax.fori_loop(0, _ROWS_PER_W // 16, mod_body, 0)
        bufs = (buf0, buf1)
        sems = (sem0, sem1)
        for k in range(_NCHUNK):
            buf = bufs[k % 2]
            sem = sems[k % 2]
            pltpu.async_copy(
                table_hbm.at[idx_v.at[pl.ds(k * _CHUNK, _CHUNK)]], buf, sem
            ).wait()
            pltpu.sync_copy(
                buf.at[:, pl.ds(0, _PERIOD)],
                out_hbm.at[pl.ds(row0 + k * _CHUNK, _CHUNK)],
            )

    return gather_kernel(table, ts2)


def kernel(time_start, initial_state_params):
    ts = time_start.astype(jnp.int32).reshape(_BATCH)
    table = _build_table(initial_state_params)
    return _gather_rows(table, ts)


# speedup vs baseline: 873.9041x; 2.0830x over previous
import functools

import jax
import jax.numpy as jnp
from jax import lax
from jax.experimental import pallas as pl
from jax.experimental.pallas import tpu as pltpu
from jax.experimental.pallas import tpu_sc as plsc

_PERIOD = 168
_BATCH = 65536
_NC = 2
_NS = 16
_NW = _NC * _NS
_COLS_PER_W = _BATCH // _NW
_SUB = _PERIOD // 8
_LT = _BATCH // 128
_CT = 2
_NCHUNK = _COLS_PER_W // (_CT * 128)
_EXT = 352


def _rotations(ts, params):
    @functools.partial(
        pl.kernel,
        out_type=jax.ShapeDtypeStruct((_SUB, _LT, 8, 128), jnp.float32),
        mesh=plsc.VectorSubcoreMesh(core_axis_name="c", subcore_axis_name="s"),
        compiler_params=pltpu.CompilerParams(
            needs_layout_passes=False, use_tc_tiling_on_sc=False
        ),
        scratch_types=[
            pltpu.VMEM((_COLS_PER_W,), jnp.int32),
            pltpu.VMEM((176,), jnp.float32),
            pltpu.VMEM((_EXT,), jnp.float32),
            pltpu.VMEM((_SUB, _CT, 8, 128), jnp.float32),
            pltpu.VMEM((_SUB, _CT, 8, 128), jnp.float32),
            pltpu.SemaphoreType.DMA,
            pltpu.SemaphoreType.DMA,
        ],
    )
    def rot_kernel(ts_hbm, params_hbm, out_hbm, ts_v, params_v, ext_v,
                   vb0, vb1, sem0, sem1):
        wid = lax.axis_index("s") * _NC + lax.axis_index("c")
        col0 = wid * _COLS_PER_W
        pltpu.sync_copy(ts_hbm.at[pl.ds(col0, _COLS_PER_W)], ts_v)

        zeros = jnp.zeros((16,), jnp.float32)
        for c in range(176 // 16):
            params_v[pl.ds(c * 16, 16)] = zeros
        pltpu.sync_copy(params_hbm, params_v.at[pl.ds(0, _PERIOD - 1)])
        acc = zeros
        for c in range(176 // 16):
            acc = acc + params_v[pl.ds(c * 16, 16)]
        lanes = lax.iota(jnp.int32, 16)
        for st in (8, 4, 2, 1):
            ext_v[pl.ds(0, 16)] = acc
            acc = acc + plsc.load_gather(ext_v, [jnp.bitwise_xor(lanes, st)])
        first = -acc
        for c in range(_EXT // 16):
            j = lax.rem(c * 16 + lanes, _PERIOD)
            pidx = jnp.maximum(j - 1, 0)
            vals = plsc.load_gather(params_v, [pidx])
            ext_v[pl.ds(c * 16, 16)] = jnp.where(j == 0, first, vals)

        bufs = (vb0, vb1)
        sems = (sem0, sem1)
        pending = [(), ()]
        for chunk in range(_NCHUNK):
            b = chunk % 2
            for h in pending[b]:
                h.wait()
            vb = bufs[b]
            ccol0 = chunk * _CT * 128

            def sub_body(t, carry, vb=vb, ccol0=ccol0):
                m_vec = lax.rem(ts_v[pl.ds(ccol0 + t * 16, 16)], _PERIOD)
                ct = t // 8
                lane0 = (t % 8) * 16
                for j in range(_PERIOD):
                    vb[j // 8, ct, j % 8, pl.ds(lane0, 16)] = plsc.load_gather(
                        ext_v, [m_vec + j]
                    )
                return carry

            lax.fori_loop(0, _CT * 8, sub_body, 0)
            ct0 = wid * (_COLS_PER_W // 128) + chunk * _CT
            hs = []
            for a in range(_SUB):
                hs.append(
                    pltpu.async_copy(
                        vb.at[a], out_hbm.at[a, pl.ds(ct0, _CT)], sems[b]
                    )
                )
            pending[b] = hs
        for b in range(2):
            for h in pending[b]:
                h.wait()

    return rot_kernel(ts, params)


def kernel(time_start, initial_state_params):
    ts = time_start.astype(jnp.int32).reshape(_BATCH)
    p = _rotations(ts, initial_state_params)
    return p.transpose(0, 2, 1, 3).reshape(_PERIOD, _BATCH).T


# speedup vs baseline: 2399.3202x; 2.7455x over previous
import functools

import jax
import jax.numpy as jnp
from jax import lax
from jax.experimental import pallas as pl
from jax.experimental.pallas import tpu as pltpu
from jax.experimental.pallas import tpu_sc as plsc

_PERIOD = 168
_BATCH = 65536
_NC = 2
_NS = 16
_NW = _NC * _NS
_COLS_PER_W = _BATCH // _NW
_SUB = _PERIOD // 8
_LT = _BATCH // 128
_CT = 2
_NCHUNK = _COLS_PER_W // (_CT * 128)
_EXT = 352


def _rotations(ts, params):
    @functools.partial(
        pl.kernel,
        out_type=jax.ShapeDtypeStruct((_SUB, _LT, 8, 128), jnp.float32),
        mesh=plsc.VectorSubcoreMesh(core_axis_name="c", subcore_axis_name="s"),
        compiler_params=pltpu.CompilerParams(
            needs_layout_passes=False, use_tc_tiling_on_sc=False
        ),
        scratch_types=[
            pltpu.VMEM((_COLS_PER_W,), jnp.int32),
            pltpu.VMEM((176,), jnp.float32),
            pltpu.VMEM((_EXT,), jnp.float32),
            pltpu.VMEM((_SUB, _CT, 8, 128), jnp.float32),
            pltpu.VMEM((_SUB, _CT, 8, 128), jnp.float32),
            pltpu.SemaphoreType.DMA,
            pltpu.SemaphoreType.DMA,
        ],
    )
    def rot_kernel(ts_hbm, params_hbm, out_hbm, ts_v, params_v, ext_v,
                   vb0, vb1, sem0, sem1):
        wid = lax.axis_index("s") * _NC + lax.axis_index("c")
        col0 = wid * _COLS_PER_W
        pltpu.sync_copy(ts_hbm.at[pl.ds(col0, _COLS_PER_W)], ts_v)

        zeros = jnp.zeros((16,), jnp.float32)
        for c in range(176 // 16):
            params_v[pl.ds(c * 16, 16)] = zeros
        pltpu.sync_copy(params_hbm, params_v.at[pl.ds(0, _PERIOD - 1)])
        acc = zeros
        for c in range(176 // 16):
            acc = acc + params_v[pl.ds(c * 16, 16)]
        lanes = lax.iota(jnp.int32, 16)
        for st in (8, 4, 2, 1):
            ext_v[pl.ds(0, 16)] = acc
            acc = acc + plsc.load_gather(ext_v, [jnp.bitwise_xor(lanes, st)])
        first = -acc
        for c in range(_EXT // 16):
            j = lax.rem(c * 16 + lanes, _PERIOD)
            pidx = jnp.maximum(j - 1, 0)
            vals = plsc.load_gather(params_v, [pidx])
            ext_v[pl.ds(c * 16, 16)] = jnp.where(j == 0, first, vals)

        bufs = (vb0, vb1)
        sems = (sem0, sem1)
        pending = [(), ()]
        for chunk in range(_NCHUNK):
            b = chunk % 2
            for h in pending[b]:
                h.wait()
            vb = bufs[b]
            ccol0 = chunk * _CT * 128

            def sub_body(t, carry, vb=vb, ccol0=ccol0):
                m_vec = lax.rem(ts_v[pl.ds(ccol0 + t * 16, 16)], _PERIOD)
                ct = t // 8
                lane0 = (t % 8) * 16

                @plsc.parallel_loop(0, _PERIOD, unroll=8)
                def _jloop(j):
                    vb[j // 8, ct, j % 8, pl.ds(lane0, 16)] = plsc.load_gather(
                        ext_v, [m_vec + j]
                    )

                return carry

            lax.fori_loop(0, _CT * 8, sub_body, 0)
            ct0 = wid * (_COLS_PER_W // 128) + chunk * _CT
            hs = []
            for a in range(_SUB):
                hs.append(
                    pltpu.async_copy(
                        vb.at[a], out_hbm.at[a, pl.ds(ct0, _CT)], sems[b]
                    )
                )
            pending[b] = hs
        for b in range(2):
            for h in pending[b]:
                h.wait()

    return rot_kernel(ts, params)


def kernel(time_start, initial_state_params):
    ts = time_start.astype(jnp.int32).reshape(_BATCH)
    p = _rotations(ts, initial_state_params)
    return p.transpose(0, 2, 1, 3).reshape(_PERIOD, _BATCH).T


# speedup vs baseline: 2544.4106x; 1.0605x over previous
import functools

import jax
import jax.numpy as jnp
from jax import lax
from jax.experimental import pallas as pl
from jax.experimental.pallas import tpu as pltpu
from jax.experimental.pallas import tpu_sc as plsc

_PERIOD = 168
_BATCH = 65536
_NC = 2
_NS = 16
_NW = _NC * _NS
_COLS_PER_W = _BATCH // _NW
_SUB = _PERIOD // 8
_LT = _BATCH // 128
_CT = 2
_NCHUNK = _COLS_PER_W // (_CT * 128)
_EXT = 352


def _rotations(ts, params):
    @functools.partial(
        pl.kernel,
        out_type=jax.ShapeDtypeStruct((_SUB, _LT, 8, 128), jnp.float32),
        mesh=plsc.VectorSubcoreMesh(core_axis_name="c", subcore_axis_name="s"),
        compiler_params=pltpu.CompilerParams(
            needs_layout_passes=False, use_tc_tiling_on_sc=False
        ),
        scratch_types=[
            pltpu.VMEM((_COLS_PER_W,), jnp.int32),
            pltpu.VMEM((176,), jnp.float32),
            pltpu.VMEM((_EXT,), jnp.float32),
            pltpu.VMEM((_SUB, _CT, 8, 128), jnp.float32),
            pltpu.VMEM((_SUB, _CT, 8, 128), jnp.float32),
            pltpu.SemaphoreType.DMA,
            pltpu.SemaphoreType.DMA,
        ],
    )
    def rot_kernel(ts_hbm, params_hbm, out_hbm, ts_v, params_v, ext_v,
                   vb0, vb1, sem0, sem1):
        wid = lax.axis_index("s") * _NC + lax.axis_index("c")
        col0 = wid * _COLS_PER_W
        pltpu.sync_copy(ts_hbm.at[pl.ds(col0, _COLS_PER_W)], ts_v)

        zeros = jnp.zeros((16,), jnp.float32)
        for c in range(176 // 16):
            params_v[pl.ds(c * 16, 16)] = zeros
        pltpu.sync_copy(params_hbm, params_v.at[pl.ds(0, _PERIOD - 1)])
        acc = zeros
        for c in range(176 // 16):
            acc = acc + params_v[pl.ds(c * 16, 16)]
        lanes = lax.iota(jnp.int32, 16)
        for st in (8, 4, 2, 1):
            ext_v[pl.ds(0, 16)] = acc
            acc = acc + plsc.load_gather(ext_v, [jnp.bitwise_xor(lanes, st)])
        first = -acc
        for c in range(_EXT // 16):
            j = lax.rem(c * 16 + lanes, _PERIOD)
            pidx = jnp.maximum(j - 1, 0)
            vals = plsc.load_gather(params_v, [pidx])
            ext_v[pl.ds(c * 16, 16)] = jnp.where(j == 0, first, vals)

        bufs = (vb0, vb1)
        sems = (sem0, sem1)
        pending = [(), ()]
        for chunk in range(_NCHUNK):
            b = chunk % 2
            for h in pending[b]:
                h.wait()
            vb = bufs[b]
            ccol0 = chunk * _CT * 128

            @plsc.parallel_loop(0, _CT * 8)
            def _sub_body(t, vb=vb, ccol0=ccol0):
                m_vec = lax.rem(ts_v[pl.ds(ccol0 + t * 16, 16)], _PERIOD)
                ct = t // 8
                lane0 = (t % 8) * 16

                @plsc.parallel_loop(0, _PERIOD, unroll=8)
                def _jloop(j):
                    vb[j // 8, ct, j % 8, pl.ds(lane0, 16)] = plsc.load_gather(
                        ext_v, [m_vec + j]
                    )

            ct0 = wid * (_COLS_PER_W // 128) + chunk * _CT
            pending[b] = (
                pltpu.async_copy(vb, out_hbm.at[:, pl.ds(ct0, _CT)], sems[b]),
            )
        for b in range(2):
            for h in pending[b]:
                h.wait()

    return rot_kernel(ts, params)


def kernel(time_start, initial_state_params):
    ts = time_start.astype(jnp.int32).reshape(_BATCH)
    p = _rotations(ts, initial_state_params)
    return p.transpose(0, 2, 1, 3).reshape(_PERIOD, _BATCH).T
